# Initial kernel scaffold; baseline (speedup 1.0000x reference)
import jax
import jax.numpy as jnp
from jax.experimental import pallas as pl


def kernel(x, variable_seq, lead_time_seq, var_table, time_table):
    raise NotImplementedError("write your pallas kernel here")


# speedup vs baseline: 1.6019x; 1.6019x over previous
import functools

import jax
import jax.numpy as jnp
from jax import lax
from jax.experimental import pallas as pl
from jax.experimental.pallas import tpu as pltpu
from jax.experimental.pallas import tpu_sc as plsc

_B, _S, _D = 4, 4096, 768
_N = _B * _S
_NC, _NS = 2, 16
_NW = _NC * _NS
_TPW = _N // _NW
_T = 16
_NCHUNK = _TPW // _T
_NBUF = 2
_LANES = 16
_DREGS = _D // _LANES

_mesh = plsc.VectorSubcoreMesh(
    core_axis_name="c", subcore_axis_name="s", num_cores=_NC, num_subcores=_NS
)


@functools.partial(
    pl.kernel,
    out_type=jax.ShapeDtypeStruct((_N, _D), jnp.float32),
    mesh=_mesh,
    scratch_types=[
        pltpu.VMEM((_NCHUNK, _T), jnp.int32),
        pltpu.VMEM((_NCHUNK, _T), jnp.int32),
        pltpu.VMEM((_NBUF, _T, _D), jnp.float32),
        pltpu.VMEM((_NBUF, _T, _D), jnp.float32),
        pltpu.VMEM((_NBUF, _T, _D), jnp.float32),
        pltpu.SemaphoreType.DMA,
        pltpu.SemaphoreType.DMA,
        pltpu.SemaphoreType.DMA,
        pltpu.SemaphoreType.DMA,
    ],
)
def _sc_embed_add(x_hbm, vs_hbm, ls_hbm, vtab_hbm, ttab_hbm, out_hbm,
                  vidx, tidx, xbuf, vbuf, tbuf,
                  lsem0, lsem1, wsem0, wsem1):
    wid = lax.axis_index("s") * _NC + lax.axis_index("c")
    base = wid * _TPW
    lsems = (lsem0, lsem1)
    wsems = (wsem0, wsem1)

    pltpu.sync_copy(vs_hbm.at[wid], vidx)
    pltpu.sync_copy(ls_hbm.at[wid], tidx)

    def start_loads(j, b):
        row0 = base + j * _T
        pltpu.async_copy(x_hbm.at[pl.ds(row0, _T)], xbuf.at[b], lsems[b])
        pltpu.async_copy(vtab_hbm.at[vidx.at[j]], vbuf.at[b], lsems[b])
        pltpu.async_copy(ttab_hbm.at[tidx.at[j]], tbuf.at[b], lsems[b])

    def wait_loads(j, b):
        row0 = base + j * _T
        pltpu.make_async_copy(x_hbm.at[pl.ds(row0, _T)], xbuf.at[b], lsems[b]).wait()
        pltpu.make_async_copy(vtab_hbm.at[vidx.at[j]], vbuf.at[b], lsems[b]).wait()
        pltpu.make_async_copy(ttab_hbm.at[tidx.at[j]], tbuf.at[b], lsems[b]).wait()

    def start_wb(j, b):
        row0 = base + j * _T
        pltpu.async_copy(xbuf.at[b], out_hbm.at[pl.ds(row0, _T)], wsems[b])

    def wait_wb(j, b):
        row0 = base + j * _T
        pltpu.make_async_copy(xbuf.at[b], out_hbm.at[pl.ds(row0, _T)], wsems[b]).wait()

    def compute(b):
        def body(t, carry):
            for d in range(_DREGS):
                sl = pl.ds(d * _LANES, _LANES)
                plsc.addupdate(xbuf.at[b, t, sl], vbuf[b, t, sl] + tbuf[b, t, sl])
            return carry
        lax.fori_loop(0, _T, body, 0)

    start_loads(0, 0)
    start_loads(1, 1)

    def group(g, carry):
        for b in range(_NBUF):
            j = g * _NBUF + b
            wait_loads(j, b)
            compute(b)
            start_wb(j, b)
            @pl.when(g < _NCHUNK // _NBUF - 1)
            def _():
                wait_wb(j, b)
                start_loads(j + _NBUF, b)
        return carry

    lax.fori_loop(0, _NCHUNK // _NBUF, group, 0)

    wait_wb(_NCHUNK - 2, 0)
    wait_wb(_NCHUNK - 1, 1)


def kernel(x, variable_seq, lead_time_seq, var_table, time_table):
    xf = x.reshape(_N, _D)
    vs = variable_seq.reshape(_NW, _NCHUNK, _T).astype(jnp.int32)
    ls = lead_time_seq.reshape(_NW, _NCHUNK, _T).astype(jnp.int32)
    out = _sc_embed_add(xf, vs, ls, var_table, time_table)
    return out.reshape(_B, _S, _D)


# speedup vs baseline: 1.6433x; 1.0258x over previous
import functools

import jax
import jax.numpy as jnp
from jax import lax
from jax.experimental import pallas as pl
from jax.experimental.pallas import tpu as pltpu
from jax.experimental.pallas import tpu_sc as plsc

_B, _S, _D = 4, 4096, 768
_N = _B * _S
_NC, _NS = 2, 16
_NW = _NC * _NS
_TPW = _N // _NW
_T = 16
_NCHUNK = _TPW // _T
_NBUF = 2
_LANES = 16
_DREGS = _D // _LANES

_mesh = plsc.VectorSubcoreMesh(
    core_axis_name="c", subcore_axis_name="s", num_cores=_NC, num_subcores=_NS
)


@functools.partial(
    pl.kernel,
    out_type=jax.ShapeDtypeStruct((_N, _D), jnp.float32),
    mesh=_mesh,
    scratch_types=[
        pltpu.VMEM((_NCHUNK, _T), jnp.int32),
        pltpu.VMEM((_NCHUNK, _T), jnp.int32),
        pltpu.VMEM((_NBUF, _T, _D), jnp.float32),
        pltpu.VMEM((_NBUF, _T, _D), jnp.float32),
        pltpu.VMEM((_NBUF, _T, _D), jnp.float32),
        pltpu.VMEM((_NBUF, _T, _D), jnp.float32),
        pltpu.SemaphoreType.DMA,
        pltpu.SemaphoreType.DMA,
        pltpu.SemaphoreType.DMA,
        pltpu.SemaphoreType.DMA,
    ],
)
def _sc_embed_add(x_hbm, vs_hbm, ls_hbm, vtab_hbm, ttab_hbm, out_hbm,
                  vidx, tidx, xbuf, vbuf, tbuf, obuf,
                  lsem0, lsem1, wsem0, wsem1):
    wid = lax.axis_index("s") * _NC + lax.axis_index("c")
    base = wid * _TPW
    lsems = (lsem0, lsem1)
    wsems = (wsem0, wsem1)

    pltpu.sync_copy(vs_hbm.at[wid], vidx)
    pltpu.sync_copy(ls_hbm.at[wid], tidx)

    def start_loads(j, b):
        row0 = base + j * _T
        pltpu.async_copy(x_hbm.at[pl.ds(row0, _T)], xbuf.at[b], lsems[b])
        pltpu.async_copy(vtab_hbm.at[vidx.at[j]], vbuf.at[b], lsems[b])
        pltpu.async_copy(ttab_hbm.at[tidx.at[j]], tbuf.at[b], lsems[b])

    def wait_loads(j, b):
        row0 = base + j * _T
        pltpu.make_async_copy(x_hbm.at[pl.ds(row0, _T)], xbuf.at[b], lsems[b]).wait()
        pltpu.make_async_copy(vtab_hbm.at[vidx.at[j]], vbuf.at[b], lsems[b]).wait()
        pltpu.make_async_copy(ttab_hbm.at[tidx.at[j]], tbuf.at[b], lsems[b]).wait()

    def start_wb(j, b):
        row0 = base + j * _T
        pltpu.async_copy(obuf.at[b], out_hbm.at[pl.ds(row0, _T)], wsems[b])

    def wait_wb(j, b):
        row0 = base + j * _T
        pltpu.make_async_copy(obuf.at[b], out_hbm.at[pl.ds(row0, _T)], wsems[b]).wait()

    def compute(b):
        def body(t, carry):
            for d in range(_DREGS):
                sl = pl.ds(d * _LANES, _LANES)
                obuf[b, t, sl] = xbuf[b, t, sl] + vbuf[b, t, sl] + tbuf[b, t, sl]
            return carry
        lax.fori_loop(0, _T, body, 0)

    start_loads(0, 0)
    start_loads(1, 1)

    def group(g, carry):
        for b in range(_NBUF):
            j = g * _NBUF + b
            wait_loads(j, b)
            @pl.when(g > 0)
            def _():
                wait_wb(j - _NBUF, b)
            compute(b)
            start_wb(j, b)
            @pl.when(g < _NCHUNK // _NBUF - 1)
            def _():
                start_loads(j + _NBUF, b)
        return carry

    lax.fori_loop(0, _NCHUNK // _NBUF, group, 0)

    wait_wb(_NCHUNK - 2, 0)
    wait_wb(_NCHUNK - 1, 1)


def kernel(x, variable_seq, lead_time_seq, var_table, time_table):
    xf = x.reshape(_N, _D)
    vs = variable_seq.reshape(_NW, _NCHUNK, _T).astype(jnp.int32)
    ls = lead_time_seq.reshape(_NW, _NCHUNK, _T).astype(jnp.int32)
    out = _sc_embed_add(xf, vs, ls, var_table, time_table)
    return out.reshape(_B, _S, _D)
